# Initial kernel scaffold; baseline (speedup 1.0000x reference)
import jax
import jax.numpy as jnp
from jax.experimental import pallas as pl


def kernel(x, u, pos, variables, edge_index, batch, W1, b1, W2, b2, W3, b3, W4, b4):
    raise NotImplementedError("write your pallas kernel here")


# speedup vs baseline: 12.7668x; 12.7668x over previous
import jax
import jax.numpy as jnp
from jax import lax
from jax.experimental import pallas as pl
from jax.experimental.pallas import tpu as pltpu
from jax.experimental.pallas import tpu_sc as plsc

N = 10000
E = 320000
F = 128
TW = 25

NC = 2
NS = 16
NW = NC * NS
CH = 128
NCHUNK = E // CH
CPW = -(-NCHUNK // NW)
NPAD = 10240
RPS = NPAD // NS


def _sigmoid(t):
    return 1.0 / (1.0 + jnp.exp(-t))


def _swish(t):
    return t * _sigmoid(t)


def _pre_body(x, u, pos, var, w1a, w1b, w1c, w1d, w1e, b1, preA, preB):
    uterm = (jnp.dot(u[...], w1c[...], preferred_element_type=jnp.float32)
             + pos[...] * w1d[...])
    preA[...] = (jnp.dot(x[...], w1a[...], preferred_element_type=jnp.float32)
                 + uterm + var[...] * w1e[...] + b1[...])
    preB[...] = (jnp.dot(x[...], w1b[...], preferred_element_type=jnp.float32)
                 - uterm)


def _pre_tables(x, u, pos, var, w1a, w1b, w1c, w1d, w1e, b1):
    return pl.pallas_call(
        _pre_body,
        out_shape=(jax.ShapeDtypeStruct((N, F), jnp.float32),
                   jax.ShapeDtypeStruct((N, F), jnp.float32)),
    )(x, u, pos, var, w1a, w1b, w1c, w1d, w1e, b1)


def _sc_gather_body(preA, preB, dst2, src2, out, idxd, idxs, rowsA, rowsB,
                    sem0, sem1):
    wid = lax.axis_index("s") * NC + lax.axis_index("c")

    def chunk(j, carry):
        cid = wid + NW * j

        @pl.when(cid < NCHUNK)
        def _():
            pltpu.sync_copy(dst2.at[cid], idxd)
            pltpu.sync_copy(src2.at[cid], idxs)
            ca = pltpu.async_copy(preA.at[idxd], rowsA, sem0)
            cb = pltpu.async_copy(preB.at[idxs], rowsB, sem1)
            ca.wait()
            cb.wait()

            def addrow(r, c2):
                for k in range(F // 16):
                    sl = pl.ds(k * 16, 16)
                    rowsA[r, sl] = rowsA[r, sl] + rowsB[r, sl]
                return c2

            lax.fori_loop(0, CH, addrow, 0)
            pltpu.sync_copy(rowsA, out.at[pl.ds(cid * CH, CH)])

        return carry

    lax.fori_loop(0, CPW, chunk, 0)


def _sc_gather(preA, preB, dst2, src2):
    mesh = plsc.VectorSubcoreMesh(core_axis_name="c", subcore_axis_name="s",
                                  num_cores=NC, num_subcores=NS)
    fn = pl.kernel(
        _sc_gather_body,
        out_type=jax.ShapeDtypeStruct((E, F), jnp.float32),
        mesh=mesh,
        scratch_types=[
            pltpu.VMEM((CH,), jnp.int32),
            pltpu.VMEM((CH,), jnp.int32),
            pltpu.VMEM((CH, F), jnp.float32),
            pltpu.VMEM((CH, F), jnp.float32),
            pltpu.SemaphoreType.DMA,
            pltpu.SemaphoreType.DMA,
        ],
    )
    return fn(preA, preB, dst2, src2)


BLK = 1280


def _msg_body(h, w2, b2, out):
    h1 = _swish(h[...])
    mm = jnp.dot(h1, w2[...], preferred_element_type=jnp.float32) + b2[...]
    out[...] = _swish(mm)


def _msg_mlp(h, w2, b2):
    return pl.pallas_call(
        _msg_body,
        grid=(E // BLK,),
        in_specs=[
            pl.BlockSpec((BLK, F), lambda i: (i, 0)),
            pl.BlockSpec((F, F), lambda i: (0, 0)),
            pl.BlockSpec((1, F), lambda i: (0, 0)),
        ],
        out_specs=pl.BlockSpec((BLK, F), lambda i: (i, 0)),
        out_shape=jax.ShapeDtypeStruct((E, F), jnp.float32),
    )(h, w2, b2)


def _sc_scatter_body(m, dst2, zrows, aggout, degout, idx, mrows, hist1, acc):
    c = lax.axis_index("c")
    sid = lax.axis_index("s")
    wid = sid * NC + c

    zero = jnp.zeros((16,), jnp.float32)
    ones = jnp.ones((16,), jnp.float32)

    pltpu.sync_copy(zrows, mrows)

    def zhist(r, carry):
        hist1[pl.ds(r * 16, 16)] = zero
        return carry

    lax.fori_loop(0, NPAD // 16, zhist, 0)

    for t in range(RPS // CH):
        pltpu.sync_copy(mrows, acc.at[pl.ds(sid * RPS + t * CH, CH)])
    plsc.subcore_barrier()

    def chunk(j, carry):
        cid = wid + NW * j

        @pl.when(cid < NCHUNK)
        def _():
            pltpu.sync_copy(dst2.at[cid], idx)
            pltpu.sync_copy(m.at[pl.ds(cid * CH, CH)], mrows)
            pltpu.sync_copy(mrows, acc.at[idx], add=True)
            for k in range(CH // 16):
                i16 = idx[pl.ds(k * 16, 16)]
                plsc.addupdate_scatter(hist1, [i16], ones)

        return carry

    lax.fori_loop(0, CPW, chunk, 0)
    pltpu.sync_copy(hist1, degout.at[wid])
    plsc.subcore_barrier()
    pltpu.sync_copy(acc.at[pl.ds(sid * RPS, RPS)],
                    aggout.at[c, pl.ds(sid * RPS, RPS)])


def _sc_scatter(m, dst2, zrows):
    mesh = plsc.VectorSubcoreMesh(core_axis_name="c", subcore_axis_name="s",
                                  num_cores=NC, num_subcores=NS)
    fn = pl.kernel(
        _sc_scatter_body,
        out_type=(jax.ShapeDtypeStruct((NC, NPAD, F), jnp.float32),
                  jax.ShapeDtypeStruct((NW, NPAD), jnp.float32)),
        mesh=mesh,
        scratch_types=[
            pltpu.VMEM((CH,), jnp.int32),
            pltpu.VMEM((CH, F), jnp.float32),
            pltpu.VMEM((NPAD,), jnp.float32),
            pltpu.VMEM_SHARED((NPAD, F), jnp.float32),
        ],
        compiler_params=pltpu.CompilerParams(needs_layout_passes=False),
    )
    return fn(m, dst2, zrows)


def _final_body(aggh, degt, x, var, w3x, w3a, w3v, b3, w4, b4, out):
    aggf = aggh[0] + aggh[1]
    deg = jnp.sum(degt[...], axis=1, keepdims=True)
    agg = aggf / jnp.maximum(deg, 1.0)
    t = (jnp.dot(x[...], w3x[...], preferred_element_type=jnp.float32)
         + jnp.dot(agg, w3a[...], preferred_element_type=jnp.float32)
         + var[...] * w3v[...] + b3[...])
    upd = _swish(t)
    t2 = jnp.dot(upd, w4[...], preferred_element_type=jnp.float32) + b4[...]
    o = x[...] + _swish(t2)
    mean = jnp.mean(o, axis=0, keepdims=True)
    v = jnp.mean(o * o, axis=0, keepdims=True) - mean * mean
    out[...] = (o - mean) * lax.rsqrt(v + 1e-5)


def _final(aggh, degt, x, var, w3x, w3a, w3v, b3, w4, b4):
    return pl.pallas_call(
        _final_body,
        out_shape=jax.ShapeDtypeStruct((N, F), jnp.float32),
    )(aggh, degt, x, var, w3x, w3a, w3v, b3, w4, b4)


def kernel(x, u, pos, variables, edge_index, batch, W1, b1, W2, b2, W3, b3,
           W4, b4):
    src = edge_index[0]
    dst = edge_index[1]
    dst2 = dst.reshape(NCHUNK, CH)
    src2 = src.reshape(NCHUNK, CH)

    w1a = W1[:F]
    w1b = W1[F:2 * F]
    w1c = W1[2 * F:2 * F + TW]
    w1d = W1[2 * F + TW:2 * F + TW + 1]
    w1e = W1[2 * F + TW + 1:]
    b1r = b1.reshape(1, F)
    b2r = b2.reshape(1, F)
    w3x = W3[:F]
    w3a = W3[F:2 * F]
    w3v = W3[2 * F:]
    b3r = b3.reshape(1, F)
    b4r = b4.reshape(1, F)

    preA, preB = _pre_tables(x, u, pos, variables, w1a, w1b, w1c, w1d, w1e,
                             b1r)
    h = _sc_gather(preA, preB, dst2, src2)
    m = _msg_mlp(h, W2, b2r)
    zrows = jnp.zeros((CH, F), jnp.float32)
    aggh, degp = _sc_scatter(m, dst2, zrows)
    degt = degp.T[:N]
    return _final(aggh[:, :N, :], degt, x, variables, w3x, w3a, w3v, b3r, W4,
                  b4r)


# speedup vs baseline: 17.7142x; 1.3875x over previous
import jax
import jax.numpy as jnp
from jax import lax
from jax.experimental import pallas as pl
from jax.experimental.pallas import tpu as pltpu
from jax.experimental.pallas import tpu_sc as plsc

N = 10000
E = 320000
F = 128
TW = 25

NC = 2
NS = 16
NW = NC * NS
CH = 128
NCHUNK = E // CH
CPW = -(-NCHUNK // NW)
NPAD = 10240
RPS = NPAD // NS


def _sigmoid(t):
    return 1.0 / (1.0 + jnp.exp(-t))


def _swish(t):
    return t * _sigmoid(t)


def _pre_body(x, u, pos, var, w1a, w1b, w1c, w1d, w1e, b1, preA, preB):
    uterm = (jnp.dot(u[...], w1c[...], preferred_element_type=jnp.float32)
             + pos[...] * w1d[...])
    preA[...] = (jnp.dot(x[...], w1a[...], preferred_element_type=jnp.float32)
                 + uterm + var[...] * w1e[...] + b1[...])
    preB[...] = (jnp.dot(x[...], w1b[...], preferred_element_type=jnp.float32)
                 - uterm)


def _pre_tables(x, u, pos, var, w1a, w1b, w1c, w1d, w1e, b1):
    return pl.pallas_call(
        _pre_body,
        out_shape=(jax.ShapeDtypeStruct((N, F), jnp.float32),
                   jax.ShapeDtypeStruct((N, F), jnp.float32)),
    )(x, u, pos, var, w1a, w1b, w1c, w1d, w1e, b1)


CPW2 = (CPW + 1) // 2


def _sc_gather_body(preA, preB, dstf, srcf, out, idxd1, idxs1,
                    rA0, rB0, rA1, rB1, sA0, sB0, sA1, sB1, st0, st1):
    wid = lax.axis_index("s") * NC + lax.axis_index("c")
    lo = jnp.minimum(CPW * wid, NCHUNK - CPW)

    pltpu.sync_copy(dstf.at[pl.ds(lo * CH, CPW * CH)], idxd1)
    pltpu.sync_copy(srcf.at[pl.ds(lo * CH, CPW * CH)], idxs1)

    def ichunk(ref, j):
        return ref.at[pl.ds(pl.multiple_of(j * CH, CH), CH)]

    def gather_issue(j, rA, rB, sa, sb):
        pltpu.async_copy(preA.at[ichunk(idxd1, j)], rA, sa)
        pltpu.async_copy(preB.at[ichunk(idxs1, j)], rB, sb)

    def gather_wait(j, rA, rB, sa, sb):
        pltpu.make_async_copy(preA.at[ichunk(idxd1, j)], rA, sa).wait()
        pltpu.make_async_copy(preB.at[ichunk(idxs1, j)], rB, sb).wait()

    def add_rows(rA, rB):
        def addrow(r, c2):
            for k in range(F // 16):
                sl = pl.ds(k * 16, 16)
                rA[r, sl] = rA[r, sl] + rB[r, sl]
            return c2

        lax.fori_loop(0, CH, addrow, 0)

    def store_issue(j, rA, st):
        pltpu.async_copy(rA, out.at[pl.ds((lo + j) * CH, CH)], st)

    def store_wait(rA, st):
        pltpu.make_async_copy(rA, out.at[pl.ds(0, CH)], st).wait()

    gather_issue(0, rA0, rB0, sA0, sB0)

    def pair(jj, carry):
        j0 = 2 * jj
        j1 = jnp.minimum(2 * jj + 1, CPW - 1)

        @pl.when(jj > 0)
        def _():
            store_wait(rA1, st1)

        gather_issue(j1, rA1, rB1, sA1, sB1)
        gather_wait(j0, rA0, rB0, sA0, sB0)
        add_rows(rA0, rB0)
        store_issue(j0, rA0, st0)

        gather_wait(j1, rA1, rB1, sA1, sB1)
        add_rows(rA1, rB1)

        @pl.when(jj < CPW2 - 1)
        def _():
            store_wait(rA0, st0)
            gather_issue(jnp.minimum(2 * jj + 2, CPW - 1), rA0, rB0, sA0,
                         sB0)

        store_issue(j1, rA1, st1)
        return carry

    lax.fori_loop(0, CPW2, pair, 0)
    store_wait(rA0, st0)
    store_wait(rA1, st1)


def _sc_gather(preA, preB, dstf, srcf):
    mesh = plsc.VectorSubcoreMesh(core_axis_name="c", subcore_axis_name="s",
                                  num_cores=NC, num_subcores=NS)
    fn = pl.kernel(
        _sc_gather_body,
        out_type=jax.ShapeDtypeStruct((E, F), jnp.float32),
        mesh=mesh,
        scratch_types=[
            pltpu.VMEM((CPW * CH,), jnp.int32),
            pltpu.VMEM((CPW * CH,), jnp.int32),
            pltpu.VMEM((CH, F), jnp.float32),
            pltpu.VMEM((CH, F), jnp.float32),
            pltpu.VMEM((CH, F), jnp.float32),
            pltpu.VMEM((CH, F), jnp.float32),
            pltpu.SemaphoreType.DMA,
            pltpu.SemaphoreType.DMA,
            pltpu.SemaphoreType.DMA,
            pltpu.SemaphoreType.DMA,
            pltpu.SemaphoreType.DMA,
            pltpu.SemaphoreType.DMA,
        ],
    )
    return fn(preA, preB, dstf, srcf)


BLK = 1280


def _msg_body(h, w2, b2, out):
    h1 = _swish(h[...])
    mm = jnp.dot(h1, w2[...], preferred_element_type=jnp.float32) + b2[...]
    out[...] = _swish(mm)


def _msg_mlp(h, w2, b2):
    return pl.pallas_call(
        _msg_body,
        grid=(E // BLK,),
        in_specs=[
            pl.BlockSpec((BLK, F), lambda i: (i, 0)),
            pl.BlockSpec((F, F), lambda i: (0, 0)),
            pl.BlockSpec((1, F), lambda i: (0, 0)),
        ],
        out_specs=pl.BlockSpec((BLK, F), lambda i: (i, 0)),
        out_shape=jax.ShapeDtypeStruct((E, F), jnp.float32),
    )(h, w2, b2)


def _sc_scatter_body(m, dstf, zrows, aggout, degout, idxc,
                    m0, m1, hist1, acc, l0, l1):
    c = lax.axis_index("c")
    sid = lax.axis_index("s")
    wid = sid * NC + c
    lo = (wid * NCHUNK) // NW
    cnt = ((wid + 1) * NCHUNK) // NW - lo

    zero = jnp.zeros((16,), jnp.float32)
    ones = jnp.ones((16,), jnp.float32)

    pltpu.sync_copy(zrows, m0)

    def zhist(r, carry):
        hist1[pl.ds(r * 16, 16)] = zero
        return carry

    lax.fori_loop(0, NPAD // 16, zhist, 0)

    for t in range(RPS // CH):
        pltpu.sync_copy(m0, acc.at[pl.ds(sid * RPS + t * CH, CH)])
    plsc.subcore_barrier()

    def load_issue(j, buf, sem):
        pltpu.async_copy(m.at[pl.ds((lo + j) * CH, CH)], buf, sem)

    def load_wait(buf, sem):
        pltpu.make_async_copy(m.at[pl.ds(0, CH)], buf, sem).wait()

    load_issue(0, m0, l0)

    def scat(j, buf):
        pltpu.sync_copy(dstf.at[pl.ds((lo + j) * CH, CH)], idxc)
        for k in range(CH // 16):
            i16 = idxc[pl.ds(k * 16, 16)]
            plsc.addupdate_scatter(hist1, [i16], ones)
        pltpu.sync_copy(buf, acc.at[idxc], add=True)

    def pair(jj, carry):
        j0 = 2 * jj
        j1 = 2 * jj + 1
        load_issue(jnp.minimum(j1, CPW - 1), m1, l1)
        load_wait(m0, l0)

        @pl.when(j0 < cnt)
        def _():
            scat(j0, m0)

        @pl.when(jj < CPW2 - 1)
        def _():
            load_issue(jnp.minimum(j0 + 2, CPW - 1), m0, l0)

        load_wait(m1, l1)

        @pl.when(j1 < cnt)
        def _():
            scat(j1, m1)

        return carry

    lax.fori_loop(0, CPW2, pair, 0)
    pltpu.sync_copy(hist1, degout.at[wid])
    plsc.subcore_barrier()
    pltpu.sync_copy(acc.at[pl.ds(sid * RPS, RPS)],
                    aggout.at[c, pl.ds(sid * RPS, RPS)])


def _sc_scatter(m, dstf, zrows):
    mesh = plsc.VectorSubcoreMesh(core_axis_name="c", subcore_axis_name="s",
                                  num_cores=NC, num_subcores=NS)
    fn = pl.kernel(
        _sc_scatter_body,
        out_type=(jax.ShapeDtypeStruct((NC, NPAD, F), jnp.float32),
                  jax.ShapeDtypeStruct((NW, NPAD), jnp.float32)),
        mesh=mesh,
        scratch_types=[
            pltpu.VMEM((CH,), jnp.int32),
            pltpu.VMEM((CH, F), jnp.float32),
            pltpu.VMEM((CH, F), jnp.float32),
            pltpu.VMEM((NPAD,), jnp.float32),
            pltpu.VMEM_SHARED((NPAD, F), jnp.float32),
            pltpu.SemaphoreType.DMA,
            pltpu.SemaphoreType.DMA,
        ],
        compiler_params=pltpu.CompilerParams(needs_layout_passes=False),
    )
    return fn(m, dstf, zrows)


def _final_body(aggh, degt, x, var, w3x, w3a, w3v, b3, w4, b4, out):
    aggf = aggh[0] + aggh[1]
    deg = jnp.sum(degt[...], axis=1, keepdims=True)
    agg = aggf / jnp.maximum(deg, 1.0)
    t = (jnp.dot(x[...], w3x[...], preferred_element_type=jnp.float32)
         + jnp.dot(agg, w3a[...], preferred_element_type=jnp.float32)
         + var[...] * w3v[...] + b3[...])
    upd = _swish(t)
    t2 = jnp.dot(upd, w4[...], preferred_element_type=jnp.float32) + b4[...]
    o = x[...] + _swish(t2)
    mean = jnp.mean(o, axis=0, keepdims=True)
    v = jnp.mean(o * o, axis=0, keepdims=True) - mean * mean
    out[...] = (o - mean) * lax.rsqrt(v + 1e-5)


def _final(aggh, degt, x, var, w3x, w3a, w3v, b3, w4, b4):
    return pl.pallas_call(
        _final_body,
        out_shape=jax.ShapeDtypeStruct((N, F), jnp.float32),
    )(aggh, degt, x, var, w3x, w3a, w3v, b3, w4, b4)


def kernel(x, u, pos, variables, edge_index, batch, W1, b1, W2, b2, W3, b3,
           W4, b4):
    src = edge_index[0]
    dst = edge_index[1]

    w1a = W1[:F]
    w1b = W1[F:2 * F]
    w1c = W1[2 * F:2 * F + TW]
    w1d = W1[2 * F + TW:2 * F + TW + 1]
    w1e = W1[2 * F + TW + 1:]
    b1r = b1.reshape(1, F)
    b2r = b2.reshape(1, F)
    w3x = W3[:F]
    w3a = W3[F:2 * F]
    w3v = W3[2 * F:]
    b3r = b3.reshape(1, F)
    b4r = b4.reshape(1, F)

    preA, preB = _pre_tables(x, u, pos, variables, w1a, w1b, w1c, w1d, w1e,
                             b1r)
    h = _sc_gather(preA, preB, dst, src)
    m = _msg_mlp(h, W2, b2r)
    zrows = jnp.zeros((CH, F), jnp.float32)
    aggh, degp = _sc_scatter(m, dst, zrows)
    degt = degp.T[:N]
    return _final(aggh[:, :N, :], degt, x, variables, w3x, w3a, w3v, b3r, W4,
                  b4r)


# speedup vs baseline: 21.1857x; 1.1960x over previous
import jax
import jax.numpy as jnp
from jax import lax
from jax.experimental import pallas as pl
from jax.experimental.pallas import tpu as pltpu
from jax.experimental.pallas import tpu_sc as plsc

N = 10000
E = 320000
F = 128
TW = 25

NC = 2
NS = 16
NW = NC * NS
CH = 128
NPAD = 10240
RPS = NPAD // NS

S = 4
ES = E // S
NCHUNK = ES // CH
CPW = -(-NCHUNK // NW)
CPW2 = (CPW + 1) // 2


def _sigmoid(t):
    return 1.0 / (1.0 + jnp.exp(-t))


def _swish(t):
    return t * _sigmoid(t)


def _pre_body(x, u, pos, var, w1a, w1b, w1c, w1d, w1e, b1, preA, preB):
    uterm = (jnp.dot(u[...], w1c[...], preferred_element_type=jnp.float32)
             + pos[...] * w1d[...])
    preA[...] = (jnp.dot(x[...], w1a[...], preferred_element_type=jnp.float32)
                 + uterm + var[...] * w1e[...] + b1[...])
    preB[...] = (jnp.dot(x[...], w1b[...], preferred_element_type=jnp.float32)
                 - uterm)


def _pre_tables(x, u, pos, var, w1a, w1b, w1c, w1d, w1e, b1):
    return pl.pallas_call(
        _pre_body,
        out_shape=(jax.ShapeDtypeStruct((N, F), jnp.float32),
                   jax.ShapeDtypeStruct((N, F), jnp.float32)),
    )(x, u, pos, var, w1a, w1b, w1c, w1d, w1e, b1)


def _sc_gather_body(preA, preB, dstf, srcf, out, idxd1, idxs1,
                    rA0, rB0, rA1, rB1, sA0, sB0, sA1, sB1, st0, st1):
    wid = lax.axis_index("s") * NC + lax.axis_index("c")
    lo = jnp.minimum(CPW * wid, NCHUNK - CPW)

    pltpu.sync_copy(dstf.at[pl.ds(lo * CH, CPW * CH)], idxd1)
    pltpu.sync_copy(srcf.at[pl.ds(lo * CH, CPW * CH)], idxs1)

    def ichunk(ref, j):
        return ref.at[pl.ds(pl.multiple_of(j * CH, CH), CH)]

    def gather_issue(j, rA, rB, sa, sb):
        pltpu.async_copy(preA.at[ichunk(idxd1, j)], rA, sa)
        pltpu.async_copy(preB.at[ichunk(idxs1, j)], rB, sb)

    def gather_wait(j, rA, rB, sa, sb):
        pltpu.make_async_copy(preA.at[ichunk(idxd1, j)], rA, sa).wait()
        pltpu.make_async_copy(preB.at[ichunk(idxs1, j)], rB, sb).wait()

    def add_rows(rA, rB):
        def addrow(r, c2):
            for k in range(F // 16):
                sl = pl.ds(k * 16, 16)
                rA[r, sl] = rA[r, sl] + rB[r, sl]
            return c2

        lax.fori_loop(0, CH, addrow, 0)

    def store_issue(j, rA, st):
        pltpu.async_copy(rA, out.at[pl.ds((lo + j) * CH, CH)], st)

    def store_wait(rA, st):
        pltpu.make_async_copy(rA, out.at[pl.ds(0, CH)], st).wait()

    gather_issue(0, rA0, rB0, sA0, sB0)

    def pair(jj, carry):
        j0 = 2 * jj
        j1 = jnp.minimum(2 * jj + 1, CPW - 1)

        @pl.when(jj > 0)
        def _():
            store_wait(rA1, st1)

        gather_issue(j1, rA1, rB1, sA1, sB1)
        gather_wait(j0, rA0, rB0, sA0, sB0)
        add_rows(rA0, rB0)
        store_issue(j0, rA0, st0)

        gather_wait(j1, rA1, rB1, sA1, sB1)
        add_rows(rA1, rB1)

        @pl.when(jj < CPW2 - 1)
        def _():
            store_wait(rA0, st0)
            gather_issue(jnp.minimum(2 * jj + 2, CPW - 1), rA0, rB0, sA0,
                         sB0)

        store_issue(j1, rA1, st1)
        return carry

    lax.fori_loop(0, CPW2, pair, 0)
    store_wait(rA0, st0)
    store_wait(rA1, st1)


def _sc_gather(preA, preB, dstf, srcf):
    mesh = plsc.VectorSubcoreMesh(core_axis_name="c", subcore_axis_name="s",
                                  num_cores=NC, num_subcores=NS)
    fn = pl.kernel(
        _sc_gather_body,
        out_type=jax.ShapeDtypeStruct((ES, F), jnp.float32),
        mesh=mesh,
        scratch_types=[
            pltpu.VMEM((CPW * CH,), jnp.int32),
            pltpu.VMEM((CPW * CH,), jnp.int32),
            pltpu.VMEM((CH, F), jnp.float32),
            pltpu.VMEM((CH, F), jnp.float32),
            pltpu.VMEM((CH, F), jnp.float32),
            pltpu.VMEM((CH, F), jnp.float32),
            pltpu.SemaphoreType.DMA,
            pltpu.SemaphoreType.DMA,
            pltpu.SemaphoreType.DMA,
            pltpu.SemaphoreType.DMA,
            pltpu.SemaphoreType.DMA,
            pltpu.SemaphoreType.DMA,
        ],
    )
    return fn(preA, preB, dstf, srcf)


BLK = 1600


def _msg_body(h, w2, b2, out):
    h1 = _swish(h[...])
    mm = jnp.dot(h1, w2[...], preferred_element_type=jnp.float32) + b2[...]
    out[...] = _swish(mm)


def _msg_mlp(h, w2, b2):
    return pl.pallas_call(
        _msg_body,
        grid=(ES // BLK,),
        in_specs=[
            pl.BlockSpec((BLK, F), lambda i: (i, 0)),
            pl.BlockSpec((F, F), lambda i: (0, 0)),
            pl.BlockSpec((1, F), lambda i: (0, 0)),
        ],
        out_specs=pl.BlockSpec((BLK, F), lambda i: (i, 0)),
        out_shape=jax.ShapeDtypeStruct((ES, F), jnp.float32),
    )(h, w2, b2)


def _make_scatter_body(first):
    def body(*args):
        if first:
            (m, dstf, zrows, aggout, degout, idxc, m0, m1, hist1, acc,
             l0, l1) = args
        else:
            (m, dstf, aggp, degp, aggout, degout, idxc, m0, m1, hist1, acc,
             l0, l1) = args
        c = lax.axis_index("c")
        sid = lax.axis_index("s")
        wid = sid * NC + c
        lo = (wid * NCHUNK) // NW
        cnt = ((wid + 1) * NCHUNK) // NW - lo

        zero = jnp.zeros((16,), jnp.float32)
        ones = jnp.ones((16,), jnp.float32)

        if first:
            pltpu.sync_copy(zrows, m0)

            def zhist(r, carry):
                hist1[pl.ds(r * 16, 16)] = zero
                return carry

            lax.fori_loop(0, NPAD // 16, zhist, 0)
            for t in range(RPS // CH):
                pltpu.sync_copy(m0, acc.at[pl.ds(sid * RPS + t * CH, CH)])
        else:
            pltpu.sync_copy(degp.at[wid], hist1)
            pltpu.sync_copy(aggp.at[c, pl.ds(sid * RPS, RPS)],
                            acc.at[pl.ds(sid * RPS, RPS)])
        plsc.subcore_barrier()

        def load_issue(j, buf, sem):
            pltpu.async_copy(m.at[pl.ds((lo + j) * CH, CH)], buf, sem)

        def load_wait(buf, sem):
            pltpu.make_async_copy(m.at[pl.ds(0, CH)], buf, sem).wait()

        def scat(j, buf):
            pltpu.sync_copy(dstf.at[pl.ds((lo + j) * CH, CH)], idxc)
            for k in range(CH // 16):
                i16 = idxc[pl.ds(k * 16, 16)]
                plsc.addupdate_scatter(hist1, [i16], ones)
            pltpu.sync_copy(buf, acc.at[idxc], add=True)

        load_issue(0, m0, l0)

        def pair(jj, carry):
            j0 = 2 * jj
            j1 = 2 * jj + 1
            load_issue(jnp.minimum(j1, CPW - 1), m1, l1)
            load_wait(m0, l0)

            @pl.when(j0 < cnt)
            def _():
                scat(j0, m0)

            @pl.when(jj < CPW2 - 1)
            def _():
                load_issue(jnp.minimum(j0 + 2, CPW - 1), m0, l0)

            load_wait(m1, l1)

            @pl.when(j1 < cnt)
            def _():
                scat(j1, m1)

            return carry

        lax.fori_loop(0, CPW2, pair, 0)
        pltpu.sync_copy(hist1, degout.at[wid])
        plsc.subcore_barrier()
        pltpu.sync_copy(acc.at[pl.ds(sid * RPS, RPS)],
                        aggout.at[c, pl.ds(sid * RPS, RPS)])

    return body


def _sc_scatter(m, dstf, zrows, aggp, degp, first):
    mesh = plsc.VectorSubcoreMesh(core_axis_name="c", subcore_axis_name="s",
                                  num_cores=NC, num_subcores=NS)
    fn = pl.kernel(
        _make_scatter_body(first),
        out_type=(jax.ShapeDtypeStruct((NC, NPAD, F), jnp.float32),
                  jax.ShapeDtypeStruct((NW, NPAD), jnp.float32)),
        mesh=mesh,
        scratch_types=[
            pltpu.VMEM((CH,), jnp.int32),
            pltpu.VMEM((CH, F), jnp.float32),
            pltpu.VMEM((CH, F), jnp.float32),
            pltpu.VMEM((NPAD,), jnp.float32),
            pltpu.VMEM_SHARED((NPAD, F), jnp.float32),
            pltpu.SemaphoreType.DMA,
            pltpu.SemaphoreType.DMA,
        ],
        compiler_params=pltpu.CompilerParams(needs_layout_passes=False),
    )
    if first:
        return fn(m, dstf, zrows)
    return fn(m, dstf, aggp, degp)


def _final_body(aggh, degt, x, var, w3x, w3a, w3v, b3, w4, b4, out):
    aggf = aggh[0] + aggh[1]
    deg = jnp.sum(degt[...], axis=1, keepdims=True)
    agg = aggf / jnp.maximum(deg, 1.0)
    t = (jnp.dot(x[...], w3x[...], preferred_element_type=jnp.float32)
         + jnp.dot(agg, w3a[...], preferred_element_type=jnp.float32)
         + var[...] * w3v[...] + b3[...])
    upd = _swish(t)
    t2 = jnp.dot(upd, w4[...], preferred_element_type=jnp.float32) + b4[...]
    o = x[...] + _swish(t2)
    mean = jnp.mean(o, axis=0, keepdims=True)
    v = jnp.mean(o * o, axis=0, keepdims=True) - mean * mean
    out[...] = (o - mean) * lax.rsqrt(v + 1e-5)


def _final(aggh, degt, x, var, w3x, w3a, w3v, b3, w4, b4):
    return pl.pallas_call(
        _final_body,
        out_shape=jax.ShapeDtypeStruct((N, F), jnp.float32),
    )(aggh, degt, x, var, w3x, w3a, w3v, b3, w4, b4)


def kernel(x, u, pos, variables, edge_index, batch, W1, b1, W2, b2, W3, b3,
           W4, b4):
    src = edge_index[0]
    dst = edge_index[1]

    w1a = W1[:F]
    w1b = W1[F:2 * F]
    w1c = W1[2 * F:2 * F + TW]
    w1d = W1[2 * F + TW:2 * F + TW + 1]
    w1e = W1[2 * F + TW + 1:]
    b1r = b1.reshape(1, F)
    b2r = b2.reshape(1, F)
    w3x = W3[:F]
    w3a = W3[F:2 * F]
    w3v = W3[2 * F:]
    b3r = b3.reshape(1, F)
    b4r = b4.reshape(1, F)

    preA, preB = _pre_tables(x, u, pos, variables, w1a, w1b, w1c, w1d, w1e,
                             b1r)
    zrows = jnp.zeros((CH, F), jnp.float32)

    aggh = degp = None
    for s in range(S):
        dsl = lax.slice(dst, (s * ES,), ((s + 1) * ES,))
        ssl = lax.slice(src, (s * ES,), ((s + 1) * ES,))
        h = _sc_gather(preA, preB, dsl, ssl)
        m = _msg_mlp(h, W2, b2r)
        aggh, degp = _sc_scatter(m, dsl, zrows, aggh, degp, s == 0)

    degt = degp.T[:N]
    return _final(aggh[:, :N, :], degt, x, variables, w3x, w3a, w3v, b3r, W4,
                  b4r)


# speedup vs baseline: 21.5702x; 1.0181x over previous
import jax
import jax.numpy as jnp
from jax import lax
from jax.experimental import pallas as pl
from jax.experimental.pallas import tpu as pltpu
from jax.experimental.pallas import tpu_sc as plsc

N = 10000
E = 320000
F = 128
TW = 25

NC = 2
NS = 16
NW = NC * NS
CH = 128
NPAD = 10240
RPS = NPAD // NS

S = 4
ES = E // S
NCHUNK = ES // CH
CPW = -(-NCHUNK // NW)
CPW2 = (CPW + 1) // 2


def _sigmoid(t):
    return 1.0 / (1.0 + jnp.exp(-t))


def _swish(t):
    return t * _sigmoid(t)


def _pre_body(x, u, pos, var, w1a, w1b, w1c, w1d, w1e, b1, preA, preB):
    uterm = (jnp.dot(u[...], w1c[...], preferred_element_type=jnp.float32)
             + pos[...] * w1d[...])
    preA[...] = (jnp.dot(x[...], w1a[...], preferred_element_type=jnp.float32)
                 + uterm + var[...] * w1e[...] + b1[...])
    preB[...] = (jnp.dot(x[...], w1b[...], preferred_element_type=jnp.float32)
                 - uterm)


def _pre_tables(x, u, pos, var, w1a, w1b, w1c, w1d, w1e, b1):
    return pl.pallas_call(
        _pre_body,
        out_shape=(jax.ShapeDtypeStruct((N, F), jnp.float32),
                   jax.ShapeDtypeStruct((N, F), jnp.float32)),
    )(x, u, pos, var, w1a, w1b, w1c, w1d, w1e, b1)


def _sc_gather_body(preA, preB, dstf, srcf, out, idxd1, idxs1,
                    rA0, rB0, rA1, rB1, sA0, sB0, sA1, sB1, st0, st1):
    wid = lax.axis_index("s") * NC + lax.axis_index("c")
    lo = jnp.minimum(CPW * wid, NCHUNK - CPW)

    pltpu.sync_copy(dstf.at[pl.ds(lo * CH, CPW * CH)], idxd1)
    pltpu.sync_copy(srcf.at[pl.ds(lo * CH, CPW * CH)], idxs1)

    def ichunk(ref, j):
        return ref.at[pl.ds(pl.multiple_of(j * CH, CH), CH)]

    def gather_issue(j, rA, rB, sa, sb):
        pltpu.async_copy(preA.at[ichunk(idxd1, j)], rA, sa)
        pltpu.async_copy(preB.at[ichunk(idxs1, j)], rB, sb)

    def gather_wait(j, rA, rB, sa, sb):
        pltpu.make_async_copy(preA.at[ichunk(idxd1, j)], rA, sa).wait()
        pltpu.make_async_copy(preB.at[ichunk(idxs1, j)], rB, sb).wait()

    def add_rows(rA, rB):
        @plsc.parallel_loop(0, CH, 1, unroll=4)
        def _(r):
            for k in range(F // 16):
                sl = pl.ds(k * 16, 16)
                rA[r, sl] = rA[r, sl] + rB[r, sl]

    def store_issue(j, rA, st):
        pltpu.async_copy(rA, out.at[pl.ds((lo + j) * CH, CH)], st)

    def store_wait(rA, st):
        pltpu.make_async_copy(rA, out.at[pl.ds(0, CH)], st).wait()

    gather_issue(0, rA0, rB0, sA0, sB0)

    def pair(jj, carry):
        j0 = 2 * jj
        j1 = jnp.minimum(2 * jj + 1, CPW - 1)

        @pl.when(jj > 0)
        def _():
            store_wait(rA1, st1)

        gather_issue(j1, rA1, rB1, sA1, sB1)
        gather_wait(j0, rA0, rB0, sA0, sB0)
        add_rows(rA0, rB0)
        store_issue(j0, rA0, st0)

        gather_wait(j1, rA1, rB1, sA1, sB1)
        add_rows(rA1, rB1)

        @pl.when(jj < CPW2 - 1)
        def _():
            store_wait(rA0, st0)
            gather_issue(jnp.minimum(2 * jj + 2, CPW - 1), rA0, rB0, sA0,
                         sB0)

        store_issue(j1, rA1, st1)
        return carry

    lax.fori_loop(0, CPW2, pair, 0)
    store_wait(rA0, st0)
    store_wait(rA1, st1)


def _sc_gather(preA, preB, dstf, srcf):
    mesh = plsc.VectorSubcoreMesh(core_axis_name="c", subcore_axis_name="s",
                                  num_cores=NC, num_subcores=NS)
    fn = pl.kernel(
        _sc_gather_body,
        out_type=jax.ShapeDtypeStruct((ES, F), jnp.float32),
        mesh=mesh,
        scratch_types=[
            pltpu.VMEM((CPW * CH,), jnp.int32),
            pltpu.VMEM((CPW * CH,), jnp.int32),
            pltpu.VMEM((CH, F), jnp.float32),
            pltpu.VMEM((CH, F), jnp.float32),
            pltpu.VMEM((CH, F), jnp.float32),
            pltpu.VMEM((CH, F), jnp.float32),
            pltpu.SemaphoreType.DMA,
            pltpu.SemaphoreType.DMA,
            pltpu.SemaphoreType.DMA,
            pltpu.SemaphoreType.DMA,
            pltpu.SemaphoreType.DMA,
            pltpu.SemaphoreType.DMA,
        ],
    )
    return fn(preA, preB, dstf, srcf)


BLK = 1600


def _msg_body(h, w2, b2, out):
    h1 = _swish(h[...])
    mm = jnp.dot(h1, w2[...], preferred_element_type=jnp.float32) + b2[...]
    out[...] = _swish(mm)


def _msg_mlp(h, w2, b2):
    return pl.pallas_call(
        _msg_body,
        grid=(ES // BLK,),
        in_specs=[
            pl.BlockSpec((BLK, F), lambda i: (i, 0)),
            pl.BlockSpec((F, F), lambda i: (0, 0)),
            pl.BlockSpec((1, F), lambda i: (0, 0)),
        ],
        out_specs=pl.BlockSpec((BLK, F), lambda i: (i, 0)),
        out_shape=jax.ShapeDtypeStruct((ES, F), jnp.float32),
    )(h, w2, b2)


def _make_scatter_body(first):
    def body(*args):
        if first:
            (m, dstf, zrows, aggout, degout, i0, i1, m0, m1, hist1, acc,
             l0, l1, si0, si1, sc0, sc1) = args
        else:
            (m, dstf, aggp, degp, aggout, degout, i0, i1, m0, m1, hist1,
             acc, l0, l1, si0, si1, sc0, sc1) = args
        c = lax.axis_index("c")
        sid = lax.axis_index("s")
        wid = sid * NC + c
        lo = (wid * NCHUNK) // NW
        cnt = ((wid + 1) * NCHUNK) // NW - lo

        zero = jnp.zeros((16,), jnp.float32)
        ones = jnp.ones((16,), jnp.float32)

        if first:
            pltpu.sync_copy(zrows, m0)

            def zhist(r, carry):
                hist1[pl.ds(r * 16, 16)] = zero
                return carry

            lax.fori_loop(0, NPAD // 16, zhist, 0)
            for t in range(RPS // CH):
                pltpu.sync_copy(m0, acc.at[pl.ds(sid * RPS + t * CH, CH)])
        else:
            pltpu.sync_copy(degp.at[wid], hist1)
            pltpu.sync_copy(aggp.at[c, pl.ds(sid * RPS, RPS)],
                            acc.at[pl.ds(sid * RPS, RPS)])
        plsc.subcore_barrier()

        def load_issue(j, buf, sem):
            pltpu.async_copy(m.at[pl.ds((lo + j) * CH, CH)], buf, sem)

        def load_wait(buf, sem):
            pltpu.make_async_copy(m.at[pl.ds(0, CH)], buf, sem).wait()

        def idx_issue(j, ic, sem):
            pltpu.async_copy(dstf.at[pl.ds((lo + j) * CH, CH)], ic, sem)

        def idx_wait(ic, sem):
            pltpu.make_async_copy(dstf.at[pl.ds(0, CH)], ic, sem).wait()

        def hist_add(ic):
            for k in range(CH // 16):
                i16 = ic[pl.ds(k * 16, 16)]
                plsc.addupdate_scatter(hist1, [i16], ones)

        def scat_issue(buf, ic, sem):
            pltpu.async_copy(buf, acc.at[ic], sem, add=True)

        def scat_wait(buf, ic, sem):
            pltpu.make_async_copy(buf, acc.at[ic], sem).wait()

        load_issue(0, m0, l0)
        idx_issue(0, i0, si0)

        def pair(jj, carry):
            j0 = 2 * jj
            j1 = 2 * jj + 1

            @pl.when(jnp.logical_and(jj > 0, j0 - 1 < cnt))
            def _():
                scat_wait(m1, i1, sc1)

            load_issue(jnp.minimum(j1, CPW - 1), m1, l1)
            idx_issue(jnp.minimum(j1, CPW - 1), i1, si1)
            load_wait(m0, l0)
            idx_wait(i0, si0)

            @pl.when(j0 < cnt)
            def _():
                hist_add(i0)
                scat_issue(m0, i0, sc0)

            @pl.when(jj < CPW2 - 1)
            def _():
                @pl.when(j0 < cnt)
                def _():
                    scat_wait(m0, i0, sc0)

                load_issue(jnp.minimum(j0 + 2, CPW - 1), m0, l0)
                idx_issue(jnp.minimum(j0 + 2, CPW - 1), i0, si0)

            load_wait(m1, l1)
            idx_wait(i1, si1)

            @pl.when(j1 < cnt)
            def _():
                hist_add(i1)
                scat_issue(m1, i1, sc1)

            return carry

        lax.fori_loop(0, CPW2, pair, 0)

        @pl.when(2 * (CPW2 - 1) < cnt)
        def _():
            scat_wait(m0, i0, sc0)

        @pl.when(2 * CPW2 - 1 < cnt)
        def _():
            scat_wait(m1, i1, sc1)
        pltpu.sync_copy(hist1, degout.at[wid])
        plsc.subcore_barrier()
        pltpu.sync_copy(acc.at[pl.ds(sid * RPS, RPS)],
                        aggout.at[c, pl.ds(sid * RPS, RPS)])

    return body


def _sc_scatter(m, dstf, zrows, aggp, degp, first):
    mesh = plsc.VectorSubcoreMesh(core_axis_name="c", subcore_axis_name="s",
                                  num_cores=NC, num_subcores=NS)
    fn = pl.kernel(
        _make_scatter_body(first),
        out_type=(jax.ShapeDtypeStruct((NC, NPAD, F), jnp.float32),
                  jax.ShapeDtypeStruct((NW, NPAD), jnp.float32)),
        mesh=mesh,
        scratch_types=[
            pltpu.VMEM((CH,), jnp.int32),
            pltpu.VMEM((CH,), jnp.int32),
            pltpu.VMEM((CH, F), jnp.float32),
            pltpu.VMEM((CH, F), jnp.float32),
            pltpu.VMEM((NPAD,), jnp.float32),
            pltpu.VMEM_SHARED((NPAD, F), jnp.float32),
            pltpu.SemaphoreType.DMA,
            pltpu.SemaphoreType.DMA,
            pltpu.SemaphoreType.DMA,
            pltpu.SemaphoreType.DMA,
            pltpu.SemaphoreType.DMA,
            pltpu.SemaphoreType.DMA,
        ],
        compiler_params=pltpu.CompilerParams(needs_layout_passes=False),
    )
    if first:
        return fn(m, dstf, zrows)
    return fn(m, dstf, aggp, degp)


def _final_body(aggh, degt, x, var, w3x, w3a, w3v, b3, w4, b4, out):
    aggf = aggh[0] + aggh[1]
    deg = jnp.sum(degt[...], axis=1, keepdims=True)
    agg = aggf / jnp.maximum(deg, 1.0)
    t = (jnp.dot(x[...], w3x[...], preferred_element_type=jnp.float32)
         + jnp.dot(agg, w3a[...], preferred_element_type=jnp.float32)
         + var[...] * w3v[...] + b3[...])
    upd = _swish(t)
    t2 = jnp.dot(upd, w4[...], preferred_element_type=jnp.float32) + b4[...]
    o = x[...] + _swish(t2)
    mean = jnp.mean(o, axis=0, keepdims=True)
    v = jnp.mean(o * o, axis=0, keepdims=True) - mean * mean
    out[...] = (o - mean) * lax.rsqrt(v + 1e-5)


def _final(aggh, degt, x, var, w3x, w3a, w3v, b3, w4, b4):
    return pl.pallas_call(
        _final_body,
        out_shape=jax.ShapeDtypeStruct((N, F), jnp.float32),
    )(aggh, degt, x, var, w3x, w3a, w3v, b3, w4, b4)


def kernel(x, u, pos, variables, edge_index, batch, W1, b1, W2, b2, W3, b3,
           W4, b4):
    src = edge_index[0]
    dst = edge_index[1]

    w1a = W1[:F]
    w1b = W1[F:2 * F]
    w1c = W1[2 * F:2 * F + TW]
    w1d = W1[2 * F + TW:2 * F + TW + 1]
    w1e = W1[2 * F + TW + 1:]
    b1r = b1.reshape(1, F)
    b2r = b2.reshape(1, F)
    w3x = W3[:F]
    w3a = W3[F:2 * F]
    w3v = W3[2 * F:]
    b3r = b3.reshape(1, F)
    b4r = b4.reshape(1, F)

    preA, preB = _pre_tables(x, u, pos, variables, w1a, w1b, w1c, w1d, w1e,
                             b1r)
    zrows = jnp.zeros((CH, F), jnp.float32)

    aggh = degp = None
    for s in range(S):
        dsl = lax.slice(dst, (s * ES,), ((s + 1) * ES,))
        ssl = lax.slice(src, (s * ES,), ((s + 1) * ES,))
        h = _sc_gather(preA, preB, dsl, ssl)
        m = _msg_mlp(h, W2, b2r)
        aggh, degp = _sc_scatter(m, dsl, zrows, aggh, degp, s == 0)

    degt = degp.T[:N]
    return _final(aggh[:, :N, :], degt, x, variables, w3x, w3a, w3v, b3r, W4,
                  b4r)


# speedup vs baseline: 22.5688x; 1.0463x over previous
import jax
import jax.numpy as jnp
from jax import lax
from jax.experimental import pallas as pl
from jax.experimental.pallas import tpu as pltpu
from jax.experimental.pallas import tpu_sc as plsc

N = 10000
E = 320000
F = 128
TW = 25

NC = 2
NS = 16
NW = NC * NS
CH = 128
NPAD = 10240
RPS = NPAD // NS

S = 4
ES = E // S
NCHUNK = ES // CH
CPW = -(-NCHUNK // NW)
CPW2 = (CPW + 1) // 2


def _sigmoid(t):
    return 1.0 / (1.0 + jnp.exp(-t))


def _swish(t):
    return t * _sigmoid(t)


def _pre_body(x, u, pos, var, w1a, w1b, w1c, w1d, w1e, b1, preA, preB):
    uterm = (jnp.dot(u[...], w1c[...], preferred_element_type=jnp.float32)
             + pos[...] * w1d[...])
    preA[...] = (jnp.dot(x[...], w1a[...], preferred_element_type=jnp.float32)
                 + uterm + var[...] * w1e[...] + b1[...])
    preB[...] = (jnp.dot(x[...], w1b[...], preferred_element_type=jnp.float32)
                 - uterm)


def _pre_tables(x, u, pos, var, w1a, w1b, w1c, w1d, w1e, b1):
    return pl.pallas_call(
        _pre_body,
        out_shape=(jax.ShapeDtypeStruct((N, F), jnp.float32),
                   jax.ShapeDtypeStruct((N, F), jnp.float32)),
    )(x, u, pos, var, w1a, w1b, w1c, w1d, w1e, b1)


def _sc_gather_body(preA, preB, dstf, srcf, out, idxd1, idxs1,
                    rA0, rB0, rA1, rB1, h0, h1, sA0, sB0, sA1, sB1,
                    st0, st1):
    wid = lax.axis_index("s") * NC + lax.axis_index("c")
    lo = jnp.minimum(CPW * wid, NCHUNK - CPW)

    pltpu.sync_copy(dstf.at[pl.ds(lo * CH, CPW * CH)], idxd1)
    pltpu.sync_copy(srcf.at[pl.ds(lo * CH, CPW * CH)], idxs1)

    def ichunk(ref, j):
        return ref.at[pl.ds(pl.multiple_of(j * CH, CH), CH)]

    def gather_issue(j, rA, rB, sa, sb):
        pltpu.async_copy(preA.at[ichunk(idxd1, j)], rA, sa)
        pltpu.async_copy(preB.at[ichunk(idxs1, j)], rB, sb)

    def gather_wait(j, rA, rB, sa, sb):
        pltpu.make_async_copy(preA.at[ichunk(idxd1, j)], rA, sa).wait()
        pltpu.make_async_copy(preB.at[ichunk(idxs1, j)], rB, sb).wait()

    def add_rows(rA, rB, hb):
        @plsc.parallel_loop(0, CH, 1, unroll=4)
        def _(r):
            for k in range(F // 16):
                sl = pl.ds(k * 16, 16)
                hb[r, sl] = rA[r, sl] + rB[r, sl]

    def store_issue(j, hb, st):
        pltpu.async_copy(hb, out.at[pl.ds((lo + j) * CH, CH)], st)

    def store_wait(hb, st):
        pltpu.make_async_copy(hb, out.at[pl.ds(0, CH)], st).wait()

    gather_issue(0, rA0, rB0, sA0, sB0)
    gather_issue(jnp.minimum(1, CPW - 1), rA1, rB1, sA1, sB1)

    def pair(jj, carry):
        j0 = 2 * jj
        j1 = jnp.minimum(2 * jj + 1, CPW - 1)

        gather_wait(j0, rA0, rB0, sA0, sB0)

        @pl.when(jj > 0)
        def _():
            store_wait(h0, st0)

        add_rows(rA0, rB0, h0)
        store_issue(j0, h0, st0)
        gather_issue(jnp.minimum(j0 + 2, CPW - 1), rA0, rB0, sA0, sB0)

        gather_wait(j1, rA1, rB1, sA1, sB1)

        @pl.when(jj > 0)
        def _():
            store_wait(h1, st1)

        add_rows(rA1, rB1, h1)
        store_issue(j1, h1, st1)
        gather_issue(jnp.minimum(j1 + 2, CPW - 1), rA1, rB1, sA1, sB1)
        return carry

    lax.fori_loop(0, CPW2, pair, 0)
    store_wait(h0, st0)
    store_wait(h1, st1)
    gather_wait(0, rA0, rB0, sA0, sB0)
    gather_wait(0, rA1, rB1, sA1, sB1)


def _sc_gather(preA, preB, dstf, srcf):
    mesh = plsc.VectorSubcoreMesh(core_axis_name="c", subcore_axis_name="s",
                                  num_cores=NC, num_subcores=NS)
    fn = pl.kernel(
        _sc_gather_body,
        out_type=jax.ShapeDtypeStruct((ES, F), jnp.float32),
        mesh=mesh,
        scratch_types=[
            pltpu.VMEM((CPW * CH,), jnp.int32),
            pltpu.VMEM((CPW * CH,), jnp.int32),
            pltpu.VMEM((CH, F), jnp.float32),
            pltpu.VMEM((CH, F), jnp.float32),
            pltpu.VMEM((CH, F), jnp.float32),
            pltpu.VMEM((CH, F), jnp.float32),
            pltpu.VMEM((CH, F), jnp.float32),
            pltpu.VMEM((CH, F), jnp.float32),
            pltpu.SemaphoreType.DMA,
            pltpu.SemaphoreType.DMA,
            pltpu.SemaphoreType.DMA,
            pltpu.SemaphoreType.DMA,
            pltpu.SemaphoreType.DMA,
            pltpu.SemaphoreType.DMA,
        ],
    )
    return fn(preA, preB, dstf, srcf)


BLK = 1600


def _msg_body(h, w2, b2, out):
    h1 = _swish(h[...])
    mm = jnp.dot(h1, w2[...], preferred_element_type=jnp.float32) + b2[...]
    out[...] = _swish(mm)


def _msg_mlp(h, w2, b2):
    return pl.pallas_call(
        _msg_body,
        grid=(ES // BLK,),
        in_specs=[
            pl.BlockSpec((BLK, F), lambda i: (i, 0)),
            pl.BlockSpec((F, F), lambda i: (0, 0)),
            pl.BlockSpec((1, F), lambda i: (0, 0)),
        ],
        out_specs=pl.BlockSpec((BLK, F), lambda i: (i, 0)),
        out_shape=jax.ShapeDtypeStruct((ES, F), jnp.float32),
    )(h, w2, b2)


def _make_scatter_body(first):
    def body(*args):
        if first:
            (m, dstf, zrows, aggout, degout, i0, i1, m0, m1, hist1, acc,
             l0, l1, si0, si1, sc0, sc1) = args
        else:
            (m, dstf, aggp, degp, aggout, degout, i0, i1, m0, m1, hist1,
             acc, l0, l1, si0, si1, sc0, sc1) = args
        c = lax.axis_index("c")
        sid = lax.axis_index("s")
        wid = sid * NC + c
        lo = (wid * NCHUNK) // NW
        cnt = ((wid + 1) * NCHUNK) // NW - lo

        zero = jnp.zeros((16,), jnp.float32)
        ones = jnp.ones((16,), jnp.float32)

        if first:
            pltpu.sync_copy(zrows, m0)

            def zhist(r, carry):
                hist1[pl.ds(r * 16, 16)] = zero
                return carry

            lax.fori_loop(0, NPAD // 16, zhist, 0)
            for t in range(RPS // CH):
                pltpu.sync_copy(m0, acc.at[pl.ds(sid * RPS + t * CH, CH)])
        else:
            pltpu.sync_copy(degp.at[wid], hist1)
            pltpu.sync_copy(aggp.at[c, pl.ds(sid * RPS, RPS)],
                            acc.at[pl.ds(sid * RPS, RPS)])
        plsc.subcore_barrier()

        def load_issue(j, buf, sem):
            pltpu.async_copy(m.at[pl.ds((lo + j) * CH, CH)], buf, sem)

        def load_wait(buf, sem):
            pltpu.make_async_copy(m.at[pl.ds(0, CH)], buf, sem).wait()

        def idx_issue(j, ic, sem):
            pltpu.async_copy(dstf.at[pl.ds((lo + j) * CH, CH)], ic, sem)

        def idx_wait(ic, sem):
            pltpu.make_async_copy(dstf.at[pl.ds(0, CH)], ic, sem).wait()

        def hist_add(ic):
            for k in range(CH // 16):
                i16 = ic[pl.ds(k * 16, 16)]
                plsc.addupdate_scatter(hist1, [i16], ones)

        def scat_issue(buf, ic, sem):
            pltpu.async_copy(buf, acc.at[ic], sem, add=True)

        def scat_wait(buf, ic, sem):
            pltpu.make_async_copy(buf, acc.at[ic], sem).wait()

        load_issue(0, m0, l0)
        idx_issue(0, i0, si0)

        def pair(jj, carry):
            j0 = 2 * jj
            j1 = 2 * jj + 1

            @pl.when(jnp.logical_and(jj > 0, j0 - 1 < cnt))
            def _():
                scat_wait(m1, i1, sc1)

            load_issue(jnp.minimum(j1, CPW - 1), m1, l1)
            idx_issue(jnp.minimum(j1, CPW - 1), i1, si1)
            load_wait(m0, l0)
            idx_wait(i0, si0)

            @pl.when(j0 < cnt)
            def _():
                hist_add(i0)
                scat_issue(m0, i0, sc0)

            @pl.when(jj < CPW2 - 1)
            def _():
                @pl.when(j0 < cnt)
                def _():
                    scat_wait(m0, i0, sc0)

                load_issue(jnp.minimum(j0 + 2, CPW - 1), m0, l0)
                idx_issue(jnp.minimum(j0 + 2, CPW - 1), i0, si0)

            load_wait(m1, l1)
            idx_wait(i1, si1)

            @pl.when(j1 < cnt)
            def _():
                hist_add(i1)
                scat_issue(m1, i1, sc1)

            return carry

        lax.fori_loop(0, CPW2, pair, 0)

        @pl.when(2 * (CPW2 - 1) < cnt)
        def _():
            scat_wait(m0, i0, sc0)

        @pl.when(2 * CPW2 - 1 < cnt)
        def _():
            scat_wait(m1, i1, sc1)
        pltpu.sync_copy(hist1, degout.at[wid])
        plsc.subcore_barrier()
        pltpu.sync_copy(acc.at[pl.ds(sid * RPS, RPS)],
                        aggout.at[c, pl.ds(sid * RPS, RPS)])

    return body


def _sc_scatter(m, dstf, zrows, aggp, degp, first):
    mesh = plsc.VectorSubcoreMesh(core_axis_name="c", subcore_axis_name="s",
                                  num_cores=NC, num_subcores=NS)
    fn = pl.kernel(
        _make_scatter_body(first),
        out_type=(jax.ShapeDtypeStruct((NC, NPAD, F), jnp.float32),
                  jax.ShapeDtypeStruct((NW, NPAD), jnp.float32)),
        mesh=mesh,
        scratch_types=[
            pltpu.VMEM((CH,), jnp.int32),
            pltpu.VMEM((CH,), jnp.int32),
            pltpu.VMEM((CH, F), jnp.float32),
            pltpu.VMEM((CH, F), jnp.float32),
            pltpu.VMEM((NPAD,), jnp.float32),
            pltpu.VMEM_SHARED((NPAD, F), jnp.float32),
            pltpu.SemaphoreType.DMA,
            pltpu.SemaphoreType.DMA,
            pltpu.SemaphoreType.DMA,
            pltpu.SemaphoreType.DMA,
            pltpu.SemaphoreType.DMA,
            pltpu.SemaphoreType.DMA,
        ],
        compiler_params=pltpu.CompilerParams(needs_layout_passes=False),
    )
    if first:
        return fn(m, dstf, zrows)
    return fn(m, dstf, aggp, degp)


def _final_body(aggh, degt, x, var, w3x, w3a, w3v, b3, w4, b4, out):
    aggf = aggh[0, :N, :] + aggh[1, :N, :]
    deg = jnp.sum(degt[...], axis=1, keepdims=True)
    agg = aggf / jnp.maximum(deg, 1.0)
    t = (jnp.dot(x[...], w3x[...], preferred_element_type=jnp.float32)
         + jnp.dot(agg, w3a[...], preferred_element_type=jnp.float32)
         + var[...] * w3v[...] + b3[...])
    upd = _swish(t)
    t2 = jnp.dot(upd, w4[...], preferred_element_type=jnp.float32) + b4[...]
    o = x[...] + _swish(t2)
    mean = jnp.mean(o, axis=0, keepdims=True)
    v = jnp.mean(o * o, axis=0, keepdims=True) - mean * mean
    out[...] = (o - mean) * lax.rsqrt(v + 1e-5)


def _final(aggh, degt, x, var, w3x, w3a, w3v, b3, w4, b4):
    return pl.pallas_call(
        _final_body,
        out_shape=jax.ShapeDtypeStruct((N, F), jnp.float32),
    )(aggh, degt, x, var, w3x, w3a, w3v, b3, w4, b4)


def kernel(x, u, pos, variables, edge_index, batch, W1, b1, W2, b2, W3, b3,
           W4, b4):
    src = edge_index[0]
    dst = edge_index[1]

    w1a = W1[:F]
    w1b = W1[F:2 * F]
    w1c = W1[2 * F:2 * F + TW]
    w1d = W1[2 * F + TW:2 * F + TW + 1]
    w1e = W1[2 * F + TW + 1:]
    b1r = b1.reshape(1, F)
    b2r = b2.reshape(1, F)
    w3x = W3[:F]
    w3a = W3[F:2 * F]
    w3v = W3[2 * F:]
    b3r = b3.reshape(1, F)
    b4r = b4.reshape(1, F)

    preA, preB = _pre_tables(x, u, pos, variables, w1a, w1b, w1c, w1d, w1e,
                             b1r)
    zrows = jnp.zeros((CH, F), jnp.float32)

    aggh = degp = None
    for s in range(S):
        dsl = lax.slice(dst, (s * ES,), ((s + 1) * ES,))
        ssl = lax.slice(src, (s * ES,), ((s + 1) * ES,))
        h = _sc_gather(preA, preB, dsl, ssl)
        m = _msg_mlp(h, W2, b2r)
        aggh, degp = _sc_scatter(m, dsl, zrows, aggh, degp, s == 0)

    degt = degp.T[:N]
    return _final(aggh, degt, x, variables, w3x, w3a, w3v, b3r, W4, b4r)
